# baseline (speedup 1.0000x reference)
import functools

import jax
import jax.numpy as jnp
from jax.experimental import pallas as pl
from jax.experimental.pallas import tpu as pltpu

B = 2
C = 384
H = 224
W = 224
WSZ = 8
PZ = 8
NH = H // WSZ
NWIN = NH * NH
WINSZ = WSZ * WSZ
NWF = int(NWIN * 0.3)
KH = H // PZ
SCALE = float(C ** (-0.5))
G = 5
NPROG = (B * NWF) // G
CB = 128
NCB = C // CB
BAND = 8
BANDW = KH * W


def _score_body(u_ref, s_ref):
    u = u_ref[0]
    s1 = u.reshape(NH, WSZ, W).sum(axis=1)
    s_ref[0] = s1.reshape(NH, NH, WSZ).sum(axis=-1)


def _score(uncertain_map):
    return pl.pallas_call(
        _score_body,
        grid=(B,),
        in_specs=[pl.BlockSpec((1, H, W), lambda b: (b, 0, 0))],
        out_specs=pl.BlockSpec((1, NH, NH), lambda b: (b, 0, 0)),
        out_shape=jax.ShapeDtypeStruct((B, NH, NH), jnp.float32),
    )(uncertain_map)


def _t_pool_body(x_ref, o_ref, p_ref):
    t = x_ref[0].T
    o_ref[0] = t
    s = t.reshape(KH, W, CB).sum(axis=0)
    p_ref[0, 0] = s.reshape(PZ, KH, CB).sum(axis=1)


def _t_pool(fm):
    return pl.pallas_call(
        _t_pool_body,
        grid=(B, NCB, BAND),
        in_specs=[pl.BlockSpec((1, CB, BANDW), lambda b, cb, bd: (b, cb, bd))],
        out_specs=[
            pl.BlockSpec((1, BANDW, CB), lambda b, cb, bd: (b, bd, cb)),
            pl.BlockSpec((1, 1, PZ, CB), lambda b, cb, bd: (b, bd, 0, cb)),
        ],
        out_shape=[
            jax.ShapeDtypeStruct((B, H * W, C), jnp.float32),
            jax.ShapeDtypeStruct((B, BAND, PZ, C), jnp.float32),
        ],
    )(fm.reshape(B, C, H * W))


def _kv_body(p_ref, w_ref, o_ref):
    toks = p_ref[...].reshape(B, PZ * PZ, C) * (1.0 / (KH * KH))
    w = w_ref[...]
    for b in range(B):
        o_ref[b] = jnp.dot(toks[b], w, preferred_element_type=jnp.float32)


def _kv(pooled, W_kvg):
    return pl.pallas_call(
        _kv_body,
        in_specs=[pl.BlockSpec((B, BAND, PZ, C), lambda: (0, 0, 0, 0)),
                  pl.BlockSpec((C, 2 * C), lambda: (0, 0))],
        out_specs=pl.BlockSpec((B, PZ * PZ, 2 * C), lambda: (0, 0, 0)),
        out_shape=jax.ShapeDtypeStruct((B, PZ * PZ, 2 * C), jnp.float32),
    )(pooled, W_kvg)


def _softmax(logits):
    m = jnp.max(logits, axis=-1, keepdims=True)
    e = jnp.exp(logits - m)
    return e / jnp.sum(e, axis=-1, keepdims=True)


def _gelu(x):
    return 0.5 * x * (1.0 + jax.lax.erf(x * (2.0 ** -0.5)))


def _sel_mats():
    ri = jax.lax.broadcasted_iota(jnp.int32, (WINSZ, C), 0)
    ci = jax.lax.broadcasted_iota(jnp.int32, (WINSZ, C), 1)
    n = C // WINSZ
    return [(ci == n * ri + j).astype(jnp.bfloat16) for j in range(n)]


def _weird(aj, vl, sel):
    tT = jax.lax.dot_general(vl.astype(jnp.bfloat16), aj.astype(jnp.bfloat16),
                             (((0,), (1,)), ((), ())),
                             preferred_element_type=jnp.float32)
    tTb = tT.astype(jnp.bfloat16)
    return jnp.concatenate(
        [jnp.dot(r, tTb, preferred_element_type=jnp.float32) for r in sel],
        axis=1)


def _win_body(bb, wy, wx, x0, x1, x2, x3, x4, kv_ref,
              wqg, wlin0, blin0, wqkv0, wlin1, blin1, wqkv1, wproj, bproj,
              o_ref):
    bf = jnp.bfloat16
    xs = [r[...].reshape(WINSZ, C) for r in (x0, x1, x2, x3, x4)]
    x = jnp.concatenate(xs, axis=0)
    kv = kv_ref[0]
    k = kv[:, :C].astype(bf)
    v = kv[:, C:].astype(bf)
    sel = _sel_mats()
    q = jnp.dot(x.astype(bf), wqg[...], preferred_element_type=jnp.float32)
    a = _softmax(jnp.dot(q.astype(bf), k.T, preferred_element_type=jnp.float32) * SCALE)
    x = x + jnp.dot(a.astype(bf), v, preferred_element_type=jnp.float32)
    for wl, bl, wqkv in ((wlin0, blin0, wqkv0), (wlin1, blin1, wqkv1)):
        x = x + _gelu(jnp.dot(x.astype(bf), wl[...], preferred_element_type=jnp.float32) + bl[...])
        qkv = jnp.dot(x.astype(bf), wqkv[...], preferred_element_type=jnp.float32)
        adds = []
        for j in range(G):
            sl = slice(j * WINSZ, (j + 1) * WINSZ)
            ql = qkv[sl, :C]
            kl = qkv[sl, C:2 * C]
            vl = qkv[sl, 2 * C:]
            aj = _softmax(jnp.dot(ql.astype(bf), kl.T.astype(bf), preferred_element_type=jnp.float32) * SCALE)
            adds.append(_weird(aj, vl, sel))
        x = x + jnp.concatenate(adds, axis=0)
    x = x + _gelu(jnp.dot(x.astype(bf), wproj[...], preferred_element_type=jnp.float32) + bproj[...])
    o_ref[0] = x


def _win_compute(x6, kv, bb, wy, wx, W_qg, W_lin_0, b_lin_0, W_qkv_0,
                 W_lin_1, b_lin_1, W_qkv_1, W_proj, b_proj):
    def xmap(j):
        def f(i, bb, wy, wx):
            return (bb[G * i + j], wy[G * i + j], 0, wx[G * i + j], 0, 0)
        return f

    def cmap(i, bb, wy, wx):
        return (0, 0)

    grid_spec = pltpu.PrefetchScalarGridSpec(
        num_scalar_prefetch=3,
        grid=(NPROG,),
        in_specs=[
            *[pl.BlockSpec((1, 1, WSZ, 1, WSZ, C), xmap(j)) for j in range(G)],
            pl.BlockSpec((1, PZ * PZ, 2 * C), lambda i, bb, wy, wx: (bb[G * i], 0, 0)),
            pl.BlockSpec((C, C), cmap),
            pl.BlockSpec((C, C), cmap),
            pl.BlockSpec((1, C), cmap),
            pl.BlockSpec((C, 3 * C), cmap),
            pl.BlockSpec((C, C), cmap),
            pl.BlockSpec((1, C), cmap),
            pl.BlockSpec((C, 3 * C), cmap),
            pl.BlockSpec((C, C), cmap),
            pl.BlockSpec((1, C), cmap),
        ],
        out_specs=pl.BlockSpec((1, G * WINSZ, C), lambda i, bb, wy, wx: (i, 0, 0)),
    )
    bf = jnp.bfloat16
    return pl.pallas_call(
        _win_body,
        grid_spec=grid_spec,
        out_shape=jax.ShapeDtypeStruct((NPROG, G * WINSZ, C), jnp.float32),
    )(bb, wy, wx, x6, x6, x6, x6, x6, kv, W_qg.astype(bf), W_lin_0.astype(bf),
      b_lin_0.reshape(1, C), W_qkv_0.astype(bf), W_lin_1.astype(bf),
      b_lin_1.reshape(1, C), W_qkv_1.astype(bf), W_proj.astype(bf),
      b_proj.reshape(1, C))


def _scatter_body(bb, wy, wx, w_ref, x_ref, o_ref):
    o_ref[...] = w_ref[...].reshape(1, 1, WSZ, 1, WSZ, C)


def _scatter(x6, win_out, bb, wy, wx):
    grid_spec = pltpu.PrefetchScalarGridSpec(
        num_scalar_prefetch=3,
        grid=(B * NWF,),
        in_specs=[
            pl.BlockSpec((1, WSZ, WSZ, C), lambda i, bb, wy, wx: (i, 0, 0, 0)),
            pl.BlockSpec(memory_space=pltpu.MemorySpace.HBM),
        ],
        out_specs=pl.BlockSpec(
            (1, 1, WSZ, 1, WSZ, C),
            lambda i, bb, wy, wx: (bb[i], wy[i], 0, wx[i], 0, 0)),
    )
    return pl.pallas_call(
        _scatter_body,
        grid_spec=grid_spec,
        out_shape=jax.ShapeDtypeStruct((B, NH, WSZ, NH, WSZ, C), jnp.float32),
        input_output_aliases={4: 0},
    )(bb, wy, wx, win_out.reshape(B * NWF, WSZ, WSZ, C), x6)


def _t_back_body(x_ref, o_ref):
    o_ref[0] = x_ref[0].T


def _t_back(x_nhwc):
    return pl.pallas_call(
        _t_back_body,
        grid=(B, NCB, BAND),
        in_specs=[pl.BlockSpec((1, BANDW, CB), lambda b, cb, bd: (b, bd, cb))],
        out_specs=pl.BlockSpec((1, CB, BANDW), lambda b, cb, bd: (b, cb, bd)),
        out_shape=jax.ShapeDtypeStruct((B, C, H * W), jnp.float32),
    )(x_nhwc)


def kernel(feature_map, uncertain_map, W_qg, W_kvg, W_lin_0, b_lin_0,
           W_qkv_0, W_lin_1, b_lin_1, W_qkv_1, W_proj, b_proj):
    scores = _score(uncertain_map).reshape(B, NWIN)
    _, idx = jax.lax.top_k(scores, NWF)
    wy = (idx // NH).reshape(-1)
    wx = (idx % NH).reshape(-1)
    bb = jnp.repeat(jnp.arange(B, dtype=idx.dtype), NWF)

    x_nhwc, pooled = _t_pool(feature_map)
    kv = _kv(pooled, W_kvg)

    x6 = x_nhwc.reshape(B, NH, WSZ, NH, WSZ, C)
    win_out = _win_compute(x6, kv, bb, wy, wx, W_qg, W_lin_0, b_lin_0,
                           W_qkv_0, W_lin_1, b_lin_1, W_qkv_1, W_proj, b_proj)
    x6_final = _scatter(x6, win_out, bb, wy, wx)
    out = _t_back(x6_final.reshape(B, H * W, C))
    return out.reshape(B, C, H, W)


# speedup vs baseline: 1.2609x; 1.2609x over previous
import functools

import jax
import jax.numpy as jnp
from jax import lax
from jax.experimental import pallas as pl
from jax.experimental.pallas import tpu as pltpu
from jax.experimental.pallas import tpu_sc as plsc

B = 2
C = 384
H = 224
W = 224
WSZ = 8
PZ = 8
NH = H // WSZ
NWIN = NH * NH
WINSZ = WSZ * WSZ
NWF = int(NWIN * 0.3)
KH = H // PZ
SCALE = float(C ** (-0.5))
G = 5
NPROG = (B * NWF) // G
CB = 128
NCB = C // CB
BAND = 8
BANDW = KH * W


def _score_body(u_ref, s_ref):
    u = u_ref[0]
    s1 = u.reshape(NH, WSZ, W).sum(axis=1)
    s_ref[0] = s1.reshape(NH, NH, WSZ).sum(axis=-1)


def _score(uncertain_map):
    return pl.pallas_call(
        _score_body,
        grid=(B,),
        in_specs=[pl.BlockSpec((1, H, W), lambda b: (b, 0, 0))],
        out_specs=pl.BlockSpec((1, NH, NH), lambda b: (b, 0, 0)),
        out_shape=jax.ShapeDtypeStruct((B, NH, NH), jnp.float32),
    )(uncertain_map)


def _t_pool_body(x_ref, o_ref, p_ref):
    t = x_ref[0].T
    o_ref[0] = t
    s = t.reshape(KH, W, CB).sum(axis=0)
    p_ref[0, 0] = s.reshape(PZ, KH, CB).sum(axis=1)


def _t_pool(fm):
    return pl.pallas_call(
        _t_pool_body,
        grid=(B, NCB, BAND),
        in_specs=[pl.BlockSpec((1, CB, BANDW), lambda b, cb, bd: (b, cb, bd))],
        out_specs=[
            pl.BlockSpec((1, BANDW, CB), lambda b, cb, bd: (b, bd, cb)),
            pl.BlockSpec((1, 1, PZ, CB), lambda b, cb, bd: (b, bd, 0, cb)),
        ],
        out_shape=[
            jax.ShapeDtypeStruct((B, H * W, C), jnp.float32),
            jax.ShapeDtypeStruct((B, BAND, PZ, C), jnp.float32),
        ],
    )(fm.reshape(B, C, H * W))


def _kv_body(p_ref, w_ref, o_ref):
    toks = p_ref[...].reshape(B, PZ * PZ, C) * (1.0 / (KH * KH))
    w = w_ref[...]
    for b in range(B):
        o_ref[b] = jnp.dot(toks[b], w, preferred_element_type=jnp.float32)


def _kv(pooled, W_kvg):
    return pl.pallas_call(
        _kv_body,
        in_specs=[pl.BlockSpec((B, BAND, PZ, C), lambda: (0, 0, 0, 0)),
                  pl.BlockSpec((C, 2 * C), lambda: (0, 0))],
        out_specs=pl.BlockSpec((B, PZ * PZ, 2 * C), lambda: (0, 0, 0)),
        out_shape=jax.ShapeDtypeStruct((B, PZ * PZ, 2 * C), jnp.float32),
    )(pooled, W_kvg)


def _softmax(logits):
    m = jnp.max(logits, axis=-1, keepdims=True)
    e = jnp.exp(logits - m)
    return e / jnp.sum(e, axis=-1, keepdims=True)


def _gelu(x):
    return 0.5 * x * (1.0 + jax.lax.erf(x * (2.0 ** -0.5)))


def _sel_mats():
    ri = jax.lax.broadcasted_iota(jnp.int32, (WINSZ, C), 0)
    ci = jax.lax.broadcasted_iota(jnp.int32, (WINSZ, C), 1)
    n = C // WINSZ
    return [(ci == n * ri + j).astype(jnp.bfloat16) for j in range(n)]


def _weird(aj, vl, sel):
    tT = jax.lax.dot_general(vl.astype(jnp.bfloat16), aj.astype(jnp.bfloat16),
                             (((0,), (1,)), ((), ())),
                             preferred_element_type=jnp.float32)
    tTb = tT.astype(jnp.bfloat16)
    return jnp.concatenate(
        [jnp.dot(r, tTb, preferred_element_type=jnp.float32) for r in sel],
        axis=1)


def _win_body(bb, wy, wx, x0, x1, x2, x3, x4, kv_ref,
              wqg, wlin0, blin0, wqkv0, wlin1, blin1, wqkv1, wproj, bproj,
              o_ref):
    bf = jnp.bfloat16
    M = G * WINSZ
    xs = [r[...].reshape(WINSZ, C) for r in (x0, x1, x2, x3, x4)]
    x = jnp.concatenate(xs, axis=0)
    kv = kv_ref[0]
    k = kv[:, :C].astype(bf)
    v = kv[:, C:].astype(bf)
    sel = _sel_mats()
    ri = jax.lax.broadcasted_iota(jnp.int32, (M, M), 0)
    ci = jax.lax.broadcasted_iota(jnp.int32, (M, M), 1)
    amask = jnp.where(ri // WINSZ == ci // WINSZ, 0.0, -1e30)
    q = jnp.dot(x.astype(bf), wqg[...], preferred_element_type=jnp.float32)
    a = _softmax(jnp.dot(q.astype(bf), k.T, preferred_element_type=jnp.float32) * SCALE)
    x = x + jnp.dot(a.astype(bf), v, preferred_element_type=jnp.float32)
    for wl, bl, wqkv in ((wlin0, blin0, wqkv0), (wlin1, blin1, wqkv1)):
        x = x + _gelu(jnp.dot(x.astype(bf), wl[...], preferred_element_type=jnp.float32) + bl[...])
        qkv = jnp.dot(x.astype(bf), wqkv[...], preferred_element_type=jnp.float32)
        ql = qkv[:, :C].astype(bf)
        kl = qkv[:, C:2 * C].astype(bf)
        vl = qkv[:, 2 * C:].astype(bf)
        logits = jax.lax.dot_general(ql, kl, (((1,), (1,)), ((), ())),
                                     preferred_element_type=jnp.float32)
        a = _softmax(logits * SCALE + amask)
        tT_all = jax.lax.dot_general(vl, a.astype(bf), (((0,), (1,)), ((), ())),
                                     preferred_element_type=jnp.float32)
        bs = [jnp.dot(r, tT_all.astype(bf), preferred_element_type=jnp.float32)
              for r in sel]
        add = jnp.concatenate(
            [jnp.concatenate([b[:, j * WINSZ:(j + 1) * WINSZ] for b in bs],
                             axis=1) for j in range(G)], axis=0)
        x = x + add
    x = x + _gelu(jnp.dot(x.astype(bf), wproj[...], preferred_element_type=jnp.float32) + bproj[...])
    o_ref[0] = x


def _win_compute(x6, kv, bb, wy, wx, W_qg, W_lin_0, b_lin_0, W_qkv_0,
                 W_lin_1, b_lin_1, W_qkv_1, W_proj, b_proj):
    def xmap(j):
        def f(i, bb, wy, wx):
            return (bb[G * i + j], wy[G * i + j], 0, wx[G * i + j], 0, 0)
        return f

    def cmap(i, bb, wy, wx):
        return (0, 0)

    grid_spec = pltpu.PrefetchScalarGridSpec(
        num_scalar_prefetch=3,
        grid=(NPROG,),
        in_specs=[
            *[pl.BlockSpec((1, 1, WSZ, 1, WSZ, C), xmap(j)) for j in range(G)],
            pl.BlockSpec((1, PZ * PZ, 2 * C), lambda i, bb, wy, wx: (bb[G * i], 0, 0)),
            pl.BlockSpec((C, C), cmap),
            pl.BlockSpec((C, C), cmap),
            pl.BlockSpec((1, C), cmap),
            pl.BlockSpec((C, 3 * C), cmap),
            pl.BlockSpec((C, C), cmap),
            pl.BlockSpec((1, C), cmap),
            pl.BlockSpec((C, 3 * C), cmap),
            pl.BlockSpec((C, C), cmap),
            pl.BlockSpec((1, C), cmap),
        ],
        out_specs=pl.BlockSpec((1, G * WINSZ, C), lambda i, bb, wy, wx: (i, 0, 0)),
    )
    bf = jnp.bfloat16
    return pl.pallas_call(
        _win_body,
        grid_spec=grid_spec,
        out_shape=jax.ShapeDtypeStruct((NPROG, G * WINSZ, C), jnp.float32),
    )(bb, wy, wx, x6, x6, x6, x6, x6, kv, W_qg.astype(bf), W_lin_0.astype(bf),
      b_lin_0.reshape(1, C), W_qkv_0.astype(bf), W_lin_1.astype(bf),
      b_lin_1.reshape(1, C), W_qkv_1.astype(bf), W_proj.astype(bf),
      b_proj.reshape(1, C))


def _scatter_body(bb, wy, wx, w_ref, x_ref, o_ref):
    o_ref[...] = w_ref[...].reshape(1, 1, WSZ, 1, WSZ, C)


def _scatter(x6, win_out, bb, wy, wx):
    grid_spec = pltpu.PrefetchScalarGridSpec(
        num_scalar_prefetch=3,
        grid=(B * NWF,),
        in_specs=[
            pl.BlockSpec((1, WSZ, WSZ, C), lambda i, bb, wy, wx: (i, 0, 0, 0)),
            pl.BlockSpec(memory_space=pltpu.MemorySpace.HBM),
        ],
        out_specs=pl.BlockSpec(
            (1, 1, WSZ, 1, WSZ, C),
            lambda i, bb, wy, wx: (bb[i], wy[i], 0, wx[i], 0, 0)),
    )
    return pl.pallas_call(
        _scatter_body,
        grid_spec=grid_spec,
        out_shape=jax.ShapeDtypeStruct((B, NH, WSZ, NH, WSZ, C), jnp.float32),
        input_output_aliases={4: 0},
    )(bb, wy, wx, win_out.reshape(B * NWF, WSZ, WSZ, C), x6)


def _t_back_body(x_ref, o_ref):
    o_ref[0] = x_ref[0].T


def _t_back(x_nhwc):
    return pl.pallas_call(
        _t_back_body,
        grid=(B, NCB, BAND),
        in_specs=[pl.BlockSpec((1, BANDW, CB), lambda b, cb, bd: (b, bd, cb))],
        out_specs=pl.BlockSpec((1, CB, BANDW), lambda b, cb, bd: (b, cb, bd)),
        out_shape=jax.ShapeDtypeStruct((B, C, H * W), jnp.float32),
    )(x_nhwc)


def kernel(feature_map, uncertain_map, W_qg, W_kvg, W_lin_0, b_lin_0,
           W_qkv_0, W_lin_1, b_lin_1, W_qkv_1, W_proj, b_proj):
    scores = _score(uncertain_map).reshape(B, NWIN)
    _, idx = jax.lax.top_k(scores, NWF)
    wy = (idx // NH).reshape(-1)
    wx = (idx % NH).reshape(-1)
    bb = jnp.repeat(jnp.arange(B, dtype=idx.dtype), NWF)

    x_nhwc, pooled = _t_pool(feature_map)
    kv = _kv(pooled, W_kvg)

    x6 = x_nhwc.reshape(B, NH, WSZ, NH, WSZ, C)
    win_out = _win_compute(x6, kv, bb, wy, wx, W_qg, W_lin_0, b_lin_0,
                           W_qkv_0, W_lin_1, b_lin_1, W_qkv_1, W_proj, b_proj)
    x6_final = _scatter(x6, win_out, bb, wy, wx)
    out = _t_back(x6_final.reshape(B, H * W, C))
    return out.reshape(B, C, H, W)
